# Initial kernel scaffold; baseline (speedup 1.0000x reference)
import jax
import jax.numpy as jnp
from jax.experimental import pallas as pl


def kernel(keys, points, feats, w0, b0, w1, b1, w2, b2, attn_w, attn_b, f0, fb0, f1, fb1):
    raise NotImplementedError("write your pallas kernel here")


# speedup vs baseline: 9.4406x; 9.4406x over previous
import jax
import jax.numpy as jnp
from jax.experimental import pallas as pl
from jax.experimental.pallas import tpu as pltpu

B, N, K, NB, CIN, CMID, HEADS, COUT, DIM = 4, 2048, 2048, 16, 32, 16, 3, 64, 3
KB = 128
INF = 3.0e38


def _block_kernel(keys_ref, pT_ref, pf_ref, sfeats_ref,
                  w0_ref, b0_ref, w1_ref, b1_ref, w2_ref, b2_ref,
                  aw_ref, ab_ref, f0_ref, fb0_ref, f1_ref, fb1_ref,
                  out_ref):
    keys = keys_ref[0]
    pT = pT_ref[0]
    pf = pf_ref[0]

    d0 = keys[:, 0:1] - pT[0:1, :]
    d1 = keys[:, 1:2] - pT[1:2, :]
    d2 = keys[:, 2:3] - pT[2:3, :]
    dist2 = (d0 * d0 + d1 * d1) + d2 * d2

    lane = jax.lax.broadcasted_iota(jnp.int32, (KB, N), 1)

    gs = []
    for _ in range(NB):
        mn = jnp.min(dist2, axis=1, keepdims=True)
        idx = jnp.min(jnp.where(dist2 == mn, lane, N),
                      axis=1, keepdims=True)
        hit = lane == idx
        oh = hit.astype(jnp.float32)
        gs.append(jnp.dot(oh, pf, preferred_element_type=jnp.float32))
        dist2 = jnp.where(hit, INF, dist2)
    G = jnp.concatenate(gs, axis=0)

    keys_t = jnp.concatenate([keys] * NB, axis=0)
    rel = G[:, 0:DIM] - keys_t
    nf = G[:, DIM:DIM + CIN]

    h = jnp.maximum(jnp.dot(rel, w0_ref[:], preferred_element_type=jnp.float32)
                    + b0_ref[:], 0.0)
    h = jnp.maximum(jnp.dot(h, w1_ref[:], preferred_element_type=jnp.float32)
                    + b1_ref[:], 0.0)
    m = jnp.dot(h, w2_ref[:], preferred_element_type=jnp.float32) + b2_ref[:]

    sf = jnp.concatenate([sfeats_ref[0]] * NB, axis=0)
    raw = (jnp.dot(sf, aw_ref[0:CIN, :], preferred_element_type=jnp.float32)
           + jnp.dot(nf, aw_ref[CIN:2 * CIN, :], preferred_element_type=jnp.float32)
           + ab_ref[:])

    r3 = raw.reshape(NB, KB, HEADS)
    ex = jnp.exp(r3 - jnp.max(r3, axis=0, keepdims=True))
    attn = (ex / jnp.sum(ex, axis=0, keepdims=True)).reshape(NB * KB, HEADS)

    ci = jax.lax.broadcasted_iota(jnp.int32, (CMID, CMID * CIN), 0)
    qi = jax.lax.broadcasted_iota(jnp.int32, (CMID, CMID * CIN), 1)
    expand = (qi // CIN == ci).astype(jnp.float32)
    m_exp = jnp.dot(m, expand, preferred_element_type=jnp.float32)
    nf_tile = jnp.concatenate([nf] * CMID, axis=1)
    w = m_exp * nf_tile

    blk = CMID * CIN
    out1 = jnp.zeros((KB, 256), dtype=jnp.float32)
    for hh in range(HEADS):
        wa = w * attn[:, hh:hh + 1]
        s = jnp.sum(wa.reshape(NB, KB, blk), axis=0)
        out1 = out1 + jnp.dot(s, f0_ref[hh * blk:(hh + 1) * blk, :],
                              preferred_element_type=jnp.float32)

    h1 = jnp.maximum(out1 + fb0_ref[:], 0.0)
    out = jnp.dot(h1, f1_ref[:], preferred_element_type=jnp.float32) + fb1_ref[:]
    out_ref[0] = out


def kernel(keys, points, feats, w0, b0, w1, b1, w2, b2, attn_w, attn_b, f0, fb0, f1, fb1):
    pT = jnp.swapaxes(points, 1, 2)
    pf = jnp.concatenate([points, feats], axis=-1)
    b0r, b1r, b2r = b0.reshape(1, -1), b1.reshape(1, -1), b2.reshape(1, -1)
    abr, fb0r, fb1r = attn_b.reshape(1, -1), fb0.reshape(1, -1), fb1.reshape(1, -1)

    grid = (B, K // KB)
    full = lambda b, kb: (b, 0, 0)
    blkd = lambda b, kb: (b, kb, 0)
    w2d = lambda b, kb: (0, 0)

    return pl.pallas_call(
        _block_kernel,
        grid=grid,
        in_specs=[
            pl.BlockSpec((1, KB, DIM), blkd),
            pl.BlockSpec((1, DIM, N), full),
            pl.BlockSpec((1, N, DIM + CIN), full),
            pl.BlockSpec((1, KB, CIN), blkd),
            pl.BlockSpec((DIM, 32), w2d),
            pl.BlockSpec((1, 32), w2d),
            pl.BlockSpec((32, 32), w2d),
            pl.BlockSpec((1, 32), w2d),
            pl.BlockSpec((32, CMID), w2d),
            pl.BlockSpec((1, CMID), w2d),
            pl.BlockSpec((2 * CIN, HEADS), w2d),
            pl.BlockSpec((1, HEADS), w2d),
            pl.BlockSpec((CIN * CMID * HEADS, 256), w2d),
            pl.BlockSpec((1, 256), w2d),
            pl.BlockSpec((256, COUT), w2d),
            pl.BlockSpec((1, COUT), w2d),
        ],
        out_specs=pl.BlockSpec((1, KB, COUT), blkd),
        out_shape=jax.ShapeDtypeStruct((B, K, COUT), jnp.float32),
    )(keys, pT, pf, feats, w0, b0r, w1, b1r, w2, b2r,
      attn_w, abr, f0, fb0r, f1, fb1r)
